# Initial kernel scaffold; baseline (speedup 1.0000x reference)
import jax
import jax.numpy as jnp
from jax.experimental import pallas as pl


def kernel(latent, pos, edge_attr, params, edge_index):
    raise NotImplementedError("write your pallas kernel here")


# speedup vs baseline: 1.8806x; 1.8806x over previous
import functools

import jax
import jax.numpy as jnp
from jax import lax
from jax.experimental import pallas as pl
from jax.experimental.pallas import tpu as pltpu
from jax.experimental.pallas import tpu_sc as plsc

N = 10000
NP = 10240
E = 320000
HID = 128
OUT = 3
NC = 2
NS = 16
NW = NC * NS
L = 16
EW = E // NW
C = 80
NCH = EW // C
G = C // L
RPT = NP // NW // L
ZR = 64

_mesh = plsc.VectorSubcoreMesh(
    core_axis_name="c", subcore_axis_name="s", num_cores=NC, num_subcores=NS)

_f32 = jnp.float32


def _wid():
    return lax.axis_index("c") * NS + lax.axis_index("s")


@functools.partial(
    pl.kernel,
    out_type=(jax.ShapeDtypeStruct((E,), _f32),
              jax.ShapeDtypeStruct((NW * L,), _f32)),
    mesh=_mesh,
    compiler_params=pltpu.CompilerParams(needs_layout_passes=False),
    scratch_types=[
        pltpu.VMEM((NP,), _f32),
        pltpu.VMEM((NP,), _f32),
        pltpu.VMEM((NP,), _f32),
        pltpu.VMEM((HID,), _f32),
        pltpu.VMEM((HID,), _f32),
        pltpu.VMEM((HID,), _f32),
        pltpu.VMEM((HID,), _f32),
        pltpu.VMEM((C,), jnp.int32),
        pltpu.VMEM((C,), jnp.int32),
        pltpu.VMEM((C, HID), _f32),
        pltpu.VMEM((C, HID), _f32),
        pltpu.VMEM((C,), _f32),
        pltpu.VMEM((L,), _f32),
    ])
def _sc_attn(xl, xr, px, py, pz, src_e, dst_e, att_h, wex_h, wey_h, wez_h,
             alpha_o, pmax_o,
             px_v, py_v, pz_v, att_v, wex_v, wey_v, wez_v,
             src_v, dst_v, xj_v, xi_v, al_v, mx_v):
    wid = _wid()
    pltpu.sync_copy(px, px_v)
    pltpu.sync_copy(py, py_v)
    pltpu.sync_copy(pz, pz_v)
    pltpu.sync_copy(att_h, att_v)
    pltpu.sync_copy(wex_h, wex_v)
    pltpu.sync_copy(wey_h, wey_v)
    pltpu.sync_copy(wez_h, wez_v)
    base_e = wid * EW
    iot = lax.iota(jnp.int32, L)

    def chunk(j, maxv):
        off = base_e + j * C
        pltpu.sync_copy(src_e.at[pl.ds(off, C)], src_v)
        pltpu.sync_copy(dst_e.at[pl.ds(off, C)], dst_v)
        pltpu.sync_copy(xl.at[src_v], xj_v)
        pltpu.sync_copy(xr.at[dst_v], xi_v)
        rio, exs, eys, ezs = [], [], [], []
        for g in range(G):
            sg = src_v[pl.ds(g * L, L)]
            dg = dst_v[pl.ds(g * L, L)]
            exs.append(plsc.load_gather(px_v, [dg]) - plsc.load_gather(px_v, [sg]))
            eys.append(plsc.load_gather(py_v, [dg]) - plsc.load_gather(py_v, [sg]))
            ezs.append(plsc.load_gather(pz_v, [dg]) - plsc.load_gather(pz_v, [sg]))
            rio.append(g * L + iot)

        def c_step(c, accs):
            cs = pl.ds(c * L, L)
            w1c = wex_v[cs]
            w2c = wey_v[cs]
            w3c = wez_v[cs]
            afc = att_v[cs]
            out = list(accs)
            for lane in range(L):
                fvv = jnp.full((L,), 0, jnp.int32) + (c * L + lane)
                w1 = w1c[lane]
                w2 = w2c[lane]
                w3 = w3c[lane]
                af = afc[lane]
                for g in range(G):
                    xj = plsc.load_gather(xj_v, [rio[g], fvv])
                    xi = plsc.load_gather(xi_v, [rio[g], fvv])
                    v = xj + xi + w1 * exs[g] + w2 * eys[g] + w3 * ezs[g]
                    m = jnp.maximum(v, 0.2 * v)
                    out[g] = out[g] + af * m
            return tuple(out)

        accs = lax.fori_loop(
            0, HID // L, c_step, tuple(jnp.zeros((L,), _f32) for _ in range(G)))
        for g in range(G):
            al_v[pl.ds(g * L, L)] = accs[g]
            maxv = jnp.maximum(maxv, accs[g])
        pltpu.sync_copy(al_v, alpha_o.at[pl.ds(off, C)])
        return maxv

    maxv = lax.fori_loop(0, NCH, chunk, jnp.full((L,), -3.4e38, _f32))
    mx_v[...] = maxv
    pltpu.sync_copy(mx_v, pmax_o.at[pl.ds(wid * L, L)])


@functools.partial(
    pl.kernel,
    out_type=(jax.ShapeDtypeStruct((NC, NP, HID), _f32),
              jax.ShapeDtypeStruct((NC, NP), _f32)),
    mesh=_mesh,
    compiler_params=pltpu.CompilerParams(needs_layout_passes=False),
    scratch_types=[
        pltpu.VMEM_SHARED((NP, HID), _f32),
        pltpu.VMEM_SHARED((NP,), _f32),
        pltpu.VMEM((NW * L,), _f32),
        pltpu.VMEM((C,), jnp.int32),
        pltpu.VMEM((C,), jnp.int32),
        pltpu.VMEM((C, HID), _f32),
        pltpu.VMEM((C,), _f32),
        pltpu.VMEM((C,), _f32),
        pltpu.VMEM((ZR, HID), _f32),
        pltpu.VMEM((NP // NS,), _f32),
    ])
def _sc_aggr(xl, src_e, dst_e, alpha_i, pmax_i,
             outp_o, denp_o,
             out_sh, den_sh, pm_v, src_v, dst_v, xj_v, al_v, ex_v, z_v, dz_v):
    cid = lax.axis_index("c")
    sid = lax.axis_index("s")
    wid = cid * NS + sid
    zv = jnp.zeros((L,), _f32)
    for r in range(ZR):
        for c8 in range(HID // L):
            z_v[r, pl.ds(c8 * L, L)] = zv
    nds = NP // NS
    def dz_fill(i, _):
        dz_v[pl.ds(i * L, L)] = zv
        return 0
    lax.fori_loop(0, nds // L, dz_fill, 0)
    rb = sid * nds
    for t in range(nds // ZR):
        pltpu.sync_copy(z_v, out_sh.at[pl.ds(rb + t * ZR, ZR), :])
    pltpu.sync_copy(dz_v, den_sh.at[pl.ds(rb, nds)])
    plsc.subcore_barrier()

    pltpu.sync_copy(pmax_i, pm_v)
    mv = pm_v[pl.ds(0, L)]
    for i in range(1, NW * L // L):
        mv = jnp.maximum(mv, pm_v[pl.ds(i * L, L)])
    m_glob = jnp.max(mv)

    base_e = wid * EW
    iot = lax.iota(jnp.int32, L)

    def chunk(j, _):
        off = base_e + j * C
        pltpu.sync_copy(src_e.at[pl.ds(off, C)], src_v)
        pltpu.sync_copy(dst_e.at[pl.ds(off, C)], dst_v)
        pltpu.sync_copy(alpha_i.at[pl.ds(off, C)], al_v)
        exs = []
        for g in range(G):
            eg = jnp.exp(al_v[pl.ds(g * L, L)] - m_glob)
            ex_v[pl.ds(g * L, L)] = eg
            exs.append(eg)
        pltpu.sync_copy(ex_v, den_sh.at[dst_v], add=True)
        pltpu.sync_copy(xl.at[src_v], xj_v)
        rio = [g * L + iot for g in range(G)]

        def f_step(f, carry):
            fv = jnp.full((L,), 0, jnp.int32) + f
            for g in range(G):
                xv = plsc.load_gather(xj_v, [rio[g], fv])
                plsc.store_scatter(xj_v, [rio[g], fv], xv * exs[g])
            return carry

        lax.fori_loop(0, HID, f_step, 0)
        pltpu.sync_copy(xj_v, out_sh.at[dst_v], add=True)
        return 0

    lax.fori_loop(0, NCH, chunk, 0)
    plsc.subcore_barrier()
    for t in range(nds // ZR):
        pltpu.sync_copy(out_sh.at[pl.ds(rb + t * ZR, ZR), :],
                        outp_o.at[cid, pl.ds(rb + t * ZR, ZR), :])
    pltpu.sync_copy(den_sh.at[pl.ds(rb, nds)], denp_o.at[cid, pl.ds(rb, nds)])


@functools.partial(
    pl.kernel,
    out_type=jax.ShapeDtypeStruct((NP, HID), _f32),
    mesh=_mesh,
    compiler_params=pltpu.CompilerParams(needs_layout_passes=False),
    scratch_types=[
        pltpu.VMEM((HID,), _f32),
        pltpu.VMEM((L, HID), _f32),
        pltpu.VMEM((L, HID), _f32),
        pltpu.VMEM((L,), _f32),
        pltpu.VMEM((L,), _f32),
        pltpu.VMEM((L, HID), _f32),
        pltpu.VMEM((L, HID), _f32),
    ])
def _sc_norm(outp_i, denp_i, bias_h, skip_i, x_o,
             b_v, o0_v, o1_v, d0_v, d1_v, sk_v, xb_v):
    wid = _wid()
    pltpu.sync_copy(bias_h, b_v)

    def row_chunk(k, _):
        rb = (wid * RPT + k) * L
        pltpu.sync_copy(outp_i.at[0, pl.ds(rb, L), :], o0_v)
        pltpu.sync_copy(outp_i.at[1, pl.ds(rb, L), :], o1_v)
        pltpu.sync_copy(denp_i.at[0, pl.ds(rb, L)], d0_v)
        pltpu.sync_copy(denp_i.at[1, pl.ds(rb, L)], d1_v)
        pltpu.sync_copy(skip_i.at[pl.ds(rb, L), :], sk_v)
        d0c = d0_v[...]
        d1c = d1_v[...]
        for r in range(L):
            d = d0c[r] + d1c[r] + 1e-16
            for c8 in range(HID // L):
                cs = pl.ds(c8 * L, L)
                v = (o0_v[r, cs] + o1_v[r, cs]) / d + b_v[cs] + sk_v[r, cs]
                xb_v[r, cs] = jnp.where(v > 0.0, v, jnp.exp(v) - 1.0)
        pltpu.sync_copy(xb_v, x_o.at[pl.ds(rb, L), :])
        return 0

    lax.fori_loop(0, RPT, row_chunk, 0)


def _mm_body(nw):
    def body(x_ref, p_ref, wx_ref, wp_ref, b_ref, *out_refs):
        x = x_ref[...]
        p = p_ref[...]
        for i in range(nw):
            acc = jnp.dot(x, wx_ref[i], preferred_element_type=_f32)
            acc = acc + jnp.dot(p, wp_ref[i], preferred_element_type=_f32)
            out_refs[i][...] = acc + b_ref[i, 0:1, :]
    return body


def _lin(x, posp, wxs, wps, bs, nw):
    bn = 2048
    return pl.pallas_call(
        _mm_body(nw),
        grid=(NP // bn,),
        in_specs=[
            pl.BlockSpec((bn, HID), lambda i: (i, 0)),
            pl.BlockSpec((bn, HID), lambda i: (i, 0)),
            pl.BlockSpec((nw, HID, HID), lambda i: (0, 0, 0)),
            pl.BlockSpec((nw, HID, HID), lambda i: (0, 0, 0)),
            pl.BlockSpec((nw, 8, HID), lambda i: (0, 0, 0)),
        ],
        out_specs=[pl.BlockSpec((bn, HID), lambda i: (i, 0))] * nw,
        out_shape=[jax.ShapeDtypeStruct((NP, HID), _f32)] * nw,
    )(x, posp, wxs, wps, bs)


def _pad_posw(w):
    return jnp.zeros((HID, HID), _f32).at[:3, :].set(w[:, HID:].T)


def _pad_bias(b):
    return jnp.zeros((8, HID), _f32).at[0, :b.shape[0]].set(b)


def kernel(latent, pos, edge_attr, params, edge_index):
    del edge_attr
    prm = params
    src = edge_index[0]
    dst = edge_index[1]
    lat_p = jnp.pad(latent.astype(_f32), ((0, NP - N), (0, 0)))
    pos_p = jnp.pad(pos.astype(_f32), ((0, NP - N), (0, HID - 3)))
    px = jnp.pad(pos[:, 0], (0, NP - N))
    py = jnp.pad(pos[:, 1], (0, NP - N))
    pz = jnp.pad(pos[:, 2], (0, NP - N))

    zero_w = jnp.zeros((1, HID, HID), _f32)
    x = _lin(lat_p, pos_p, prm['W0'].T[None], zero_w,
             _pad_bias(prm['b0'])[None], 1)[0]

    layers = [(prm['gat0'], prm['W1'], prm['b1']),
              (prm['gat1'], prm['W2'], prm['b2']),
              (prm['gat2'], prm['W3'], prm['b3'])]
    for gat, wk, bk in layers:
        wxs = jnp.stack([gat['Wl'][:, :HID].T, gat['Wr'][:, :HID].T,
                         wk[:, :HID].T])
        wps = jnp.stack([_pad_posw(gat['Wl']), _pad_posw(gat['Wr']),
                         _pad_posw(wk)])
        bss = jnp.stack([_pad_bias(gat['bl']), _pad_bias(gat['br']),
                         _pad_bias(bk)])
        xl, xr, skip = _lin(x, pos_p, wxs, wps, bss, 3)
        alpha, pmax = _sc_attn(xl, xr, px, py, pz, src, dst, gat['att'],
                               gat['We'][:, 0], gat['We'][:, 1],
                               gat['We'][:, 2])
        outp, denp = _sc_aggr(xl, src, dst, alpha, pmax)
        x = _sc_norm(outp, denp, gat['bias'], skip)

    w4x = jnp.zeros((HID, HID), _f32).at[:, :OUT].set(prm['W4'][:, :HID].T)
    w4p = jnp.zeros((HID, HID), _f32).at[:3, :OUT].set(prm['W4'][:, HID:].T)
    out = _lin(x, pos_p, w4x[None], w4p[None], _pad_bias(prm['b4'])[None],
               1)[0]
    return out[:N, :OUT]


# speedup vs baseline: 2.2980x; 1.2220x over previous
import functools

import jax
import jax.numpy as jnp
from jax import lax
from jax.experimental import pallas as pl
from jax.experimental.pallas import tpu as pltpu
from jax.experimental.pallas import tpu_sc as plsc

N = 10000
NP = 10240
E = 320000
EP = E + 128
HID = 128
HD2 = 144
OUT = 3
NC = 2
NS = 16
NW = NC * NS
L = 16
EW = E // NW
C = 80
NCH = EW // C
G = C // L
RPT = NP // NW // L
ZR = 32

_mesh = plsc.VectorSubcoreMesh(
    core_axis_name="c", subcore_axis_name="s", num_cores=NC, num_subcores=NS)
_params = pltpu.CompilerParams(needs_layout_passes=False)

_f32 = jnp.float32


def _wid():
    return lax.axis_index("c") * NS + lax.axis_index("s")


@functools.partial(
    pl.kernel,
    out_type=(jax.ShapeDtypeStruct((EP,), _f32),
              jax.ShapeDtypeStruct((NW * L,), _f32)),
    mesh=_mesh,
    compiler_params=_params,
    scratch_types=[
        pltpu.VMEM((NP,), _f32),
        pltpu.VMEM((NP,), _f32),
        pltpu.VMEM((NP,), _f32),
        pltpu.VMEM((HID,), _f32),
        pltpu.VMEM((HID,), _f32),
        pltpu.VMEM((HID,), _f32),
        pltpu.VMEM((HID,), _f32),
        pltpu.VMEM((2, C), jnp.int32),
        pltpu.VMEM((2, C), jnp.int32),
        pltpu.VMEM((2, C, HID), _f32),
        pltpu.VMEM((2, C, HID), _f32),
        pltpu.VMEM((2, C), _f32),
        pltpu.VMEM((L,), _f32),
        pltpu.SemaphoreType.DMA,
        pltpu.SemaphoreType.DMA,
        pltpu.SemaphoreType.DMA,
        pltpu.SemaphoreType.DMA,
        pltpu.SemaphoreType.DMA,
        pltpu.SemaphoreType.DMA,
    ])
def _sc_attn(xl, xr, px, py, pz, srcp, dstp, att_h, wex_h, wey_h, wez_h,
             alpha_o, pmax_o,
             px_v, py_v, pz_v, att_v, wex_v, wey_v, wez_v,
             se_v, de_v, xj_v, xi_v, al_v, mx_v,
             s_ei0, s_ei1, s_gat0, s_gat1, s_al0, s_al1):
    wid = _wid()
    base_e = wid * EW
    s_ei = (s_ei0, s_ei1)
    s_gat = (s_gat0, s_gat1)
    s_al = (s_al0, s_al1)
    pltpu.sync_copy(px, px_v)
    pltpu.sync_copy(py, py_v)
    pltpu.sync_copy(pz, pz_v)
    pltpu.sync_copy(att_h, att_v)
    pltpu.sync_copy(wex_h, wex_v)
    pltpu.sync_copy(wey_h, wey_v)
    pltpu.sync_copy(wez_h, wez_v)
    iot = lax.iota(jnp.int32, L)
    rio = [g * L + iot for g in range(G)]

    def start_ei(j, b):
        pltpu.async_copy(srcp.at[pl.ds(base_e + j * C, C)], se_v.at[b],
                         s_ei[b])
        pltpu.async_copy(dstp.at[pl.ds(base_e + j * C, C)], de_v.at[b],
                         s_ei[b])

    def wait_ei(b):
        pltpu.make_async_copy(srcp.at[pl.ds(base_e, C)], se_v.at[b],
                              s_ei[b]).wait()
        pltpu.make_async_copy(dstp.at[pl.ds(base_e, C)], de_v.at[b],
                              s_ei[b]).wait()

    def start_gat(j, b):
        del j
        pltpu.async_copy(xl.at[se_v.at[b]], xj_v.at[b], s_gat[b])
        pltpu.async_copy(xr.at[de_v.at[b]], xi_v.at[b], s_gat[b])

    def wait_gat(b):
        pltpu.make_async_copy(xl.at[se_v.at[b]], xj_v.at[b],
                              s_gat[b]).wait()
        pltpu.make_async_copy(xr.at[de_v.at[b]], xi_v.at[b],
                              s_gat[b]).wait()

    def wait_al(b):
        pltpu.make_async_copy(al_v.at[b], alpha_o.at[pl.ds(base_e, C)],
                              s_al[b]).wait()

    def process(j, b, maxv, last):
        b1 = 1 - b

        @pl.when(j >= 2)
        def _():
            wait_al(b)

        wait_gat(b)
        if not last:
            wait_ei(b1)
            start_gat(j + 1, b1)
        sgs = [se_v[b, pl.ds(g * L, L)] for g in range(G)]
        dgs = [de_v[b, pl.ds(g * L, L)] for g in range(G)]
        exs, eys, ezs = [], [], []
        for g in range(G):
            exs.append(plsc.load_gather(px_v, [dgs[g]])
                       - plsc.load_gather(px_v, [sgs[g]]))
            eys.append(plsc.load_gather(py_v, [dgs[g]])
                       - plsc.load_gather(py_v, [sgs[g]]))
            ezs.append(plsc.load_gather(pz_v, [dgs[g]])
                       - plsc.load_gather(pz_v, [sgs[g]]))
        if not last:
            start_ei(j + 2, b)

        def c_step(c, accs):
            cs = pl.ds(c * L, L)
            w1c = wex_v[cs]
            w2c = wey_v[cs]
            w3c = wez_v[cs]
            afc = att_v[cs]
            out = list(accs)
            for lane in range(L):
                fvv = jnp.full((L,), 0, jnp.int32) + (c * L + lane)
                w1 = w1c[lane]
                w2 = w2c[lane]
                w3 = w3c[lane]
                af = afc[lane]
                for g in range(G):
                    xj = plsc.load_gather(xj_v.at[b], [rio[g], fvv])
                    xi = plsc.load_gather(xi_v.at[b], [rio[g], fvv])
                    v = xj + xi + w1 * exs[g] + w2 * eys[g] + w3 * ezs[g]
                    m = jnp.maximum(v, 0.2 * v)
                    out[g] = out[g] + af * m
            return tuple(out)

        accs = lax.fori_loop(
            0, HID // L, c_step,
            tuple(jnp.zeros((L,), _f32) for _ in range(G)))
        for g in range(G):
            al_v[b, pl.ds(g * L, L)] = accs[g]
            maxv = jnp.maximum(maxv, accs[g])
        pltpu.async_copy(al_v.at[b], alpha_o.at[pl.ds(base_e + j * C, C)],
                         s_al[b])
        return maxv

    start_ei(0, 0)
    wait_ei(0)
    start_gat(0, 0)
    start_ei(1, 1)

    def pair(i, maxv):
        j0 = i * 2
        maxv = process(j0, 0, maxv, False)
        maxv = process(j0 + 1, 1, maxv, False)
        return maxv

    maxv = lax.fori_loop(0, NCH // 2, pair,
                         jnp.full((L,), -3.4e38, _f32))
    maxv = process(NCH - 1, 0, maxv, True)
    wait_al(1)
    wait_al(0)
    wait_ei(1)
    mx_v[...] = maxv
    pltpu.sync_copy(mx_v, pmax_o.at[pl.ds(wid * L, L)])


@functools.partial(
    pl.kernel,
    out_type=(jax.ShapeDtypeStruct((NC, NP, HID), _f32),
              jax.ShapeDtypeStruct((NC, NP), _f32)),
    mesh=_mesh,
    compiler_params=_params,
    scratch_types=[
        pltpu.VMEM_SHARED((NP, HID), _f32),
        pltpu.VMEM_SHARED((NP,), _f32),
        pltpu.VMEM((NW * L,), _f32),
        pltpu.VMEM((2, C), jnp.int32),
        pltpu.VMEM((2, C), jnp.int32),
        pltpu.VMEM((2, C), jnp.int32),
        pltpu.VMEM((2, C, HID), _f32),
        pltpu.VMEM((2, C, HID), _f32),
        pltpu.VMEM((2, C), _f32),
        pltpu.VMEM((2, C), _f32),
        pltpu.VMEM((ZR, HID), _f32),
        pltpu.VMEM((NP // NS,), _f32),
        pltpu.SemaphoreType.DMA,
        pltpu.SemaphoreType.DMA,
        pltpu.SemaphoreType.DMA,
        pltpu.SemaphoreType.DMA,
        pltpu.SemaphoreType.DMA,
        pltpu.SemaphoreType.DMA,
        pltpu.SemaphoreType.DMA,
        pltpu.SemaphoreType.DMA,
        pltpu.SemaphoreType.DMA,
        pltpu.SemaphoreType.DMA,
    ])
def _sc_aggr(xl, srcp, dstp, alpha_i, pmax_i,
             outp_o, denp_o,
             out_sh, den_sh, pm_v, se_v, de_v, dsti_v, xjg_v, sc_v, ali_v,
             ex_v, z_v, dz_v,
             s_ei0, s_ei1, s_ali0, s_ali1, s_gat0, s_gat1, s_sc0, s_sc1,
             s_scx0, s_scx1):
    cid = lax.axis_index("c")
    sid = lax.axis_index("s")
    wid = cid * NS + sid
    base_e = wid * EW
    s_ei = (s_ei0, s_ei1)
    s_ali = (s_ali0, s_ali1)
    s_gat = (s_gat0, s_gat1)
    s_sc = (s_sc0, s_sc1)
    s_scx = (s_scx0, s_scx1)
    iot = lax.iota(jnp.int32, L)
    rio = [g * L + iot for g in range(G)]
    zv = jnp.zeros((L,), _f32)
    for r in range(ZR):
        for c8 in range(HID // L):
            z_v[r, pl.ds(c8 * L, L)] = zv
    nds = NP // NS

    def dz_fill(i, _):
        dz_v[pl.ds(i * L, L)] = zv
        return 0

    lax.fori_loop(0, nds // L, dz_fill, 0)
    rb = sid * nds
    for t in range(nds // ZR):
        pltpu.sync_copy(z_v, out_sh.at[pl.ds(rb + t * ZR, ZR), :])
    pltpu.sync_copy(dz_v, den_sh.at[pl.ds(rb, nds)])
    plsc.subcore_barrier()

    pltpu.sync_copy(pmax_i, pm_v)
    mv = pm_v[pl.ds(0, L)]
    for i in range(1, NW):
        mv = jnp.maximum(mv, pm_v[pl.ds(i * L, L)])
    m_glob = jnp.max(mv)

    def start_ei(j, b):
        pltpu.async_copy(srcp.at[pl.ds(base_e + j * C, C)], se_v.at[b],
                         s_ei[b])
        pltpu.async_copy(dstp.at[pl.ds(base_e + j * C, C)], de_v.at[b],
                         s_ei[b])
        pltpu.async_copy(alpha_i.at[pl.ds(base_e + j * C, C)], ali_v.at[b],
                         s_ali[b])

    def wait_ei(b):
        pltpu.make_async_copy(srcp.at[pl.ds(base_e, C)], se_v.at[b],
                              s_ei[b]).wait()
        pltpu.make_async_copy(dstp.at[pl.ds(base_e, C)], de_v.at[b],
                              s_ei[b]).wait()
        pltpu.make_async_copy(alpha_i.at[pl.ds(base_e, C)], ali_v.at[b],
                              s_ali[b]).wait()

    def start_gat(j, b):
        del j
        pltpu.async_copy(xl.at[se_v.at[b]], xjg_v.at[b], s_gat[b])

    def wait_gat(b):
        pltpu.make_async_copy(xl.at[se_v.at[b]], xjg_v.at[b],
                              s_gat[b]).wait()

    def wait_sc(b):
        pltpu.make_async_copy(sc_v.at[b], out_sh.at[dsti_v.at[b]],
                              s_sc[b]).wait()
        pltpu.make_async_copy(ex_v.at[b], den_sh.at[dsti_v.at[b]],
                              s_scx[b]).wait()

    def process(j, b, last):
        b1 = 1 - b

        @pl.when(j >= 2)
        def _():
            wait_sc(b)

        wait_gat(b)
        if not last:
            wait_ei(b1)
            start_gat(j + 1, b1)
        exs = []
        for g in range(G):
            dg = de_v[b, pl.ds(g * L, L)]
            dsti_v[b, pl.ds(g * L, L)] = dg
            eg = jnp.exp(ali_v[b, pl.ds(g * L, L)] - m_glob)
            exs.append(eg)
            ex_v[b, pl.ds(g * L, L)] = eg
        if not last:
            start_ei(j + 2, b)
        pltpu.async_copy(ex_v.at[b], den_sh.at[dsti_v.at[b]], s_scx[b],
                         add=True)

        def f_step(f, carry):
            fvv = jnp.full((L,), 0, jnp.int32) + f
            for g in range(G):
                xv = plsc.load_gather(xjg_v.at[b], [rio[g], fvv])
                plsc.store_scatter(sc_v.at[b], [rio[g], fvv], xv * exs[g])
            return carry

        lax.fori_loop(0, HID, f_step, 0)
        pltpu.async_copy(sc_v.at[b], out_sh.at[dsti_v.at[b]], s_sc[b],
                         add=True)

    start_ei(0, 0)
    wait_ei(0)
    start_gat(0, 0)
    start_ei(1, 1)

    def pair(i, _):
        j0 = i * 2
        process(j0, 0, False)
        process(j0 + 1, 1, False)
        return 0

    lax.fori_loop(0, NCH // 2, pair, 0)
    process(NCH - 1, 0, True)
    wait_sc(1)
    wait_sc(0)
    wait_ei(1)
    plsc.subcore_barrier()
    for t in range(nds // ZR):
        pltpu.sync_copy(out_sh.at[pl.ds(rb + t * ZR, ZR), :],
                        outp_o.at[cid, pl.ds(rb + t * ZR, ZR), :])
    pltpu.sync_copy(den_sh.at[pl.ds(rb, nds)], denp_o.at[cid, pl.ds(rb, nds)])


@functools.partial(
    pl.kernel,
    out_type=jax.ShapeDtypeStruct((NP, HID), _f32),
    mesh=_mesh,
    compiler_params=_params,
    scratch_types=[
        pltpu.VMEM((HID,), _f32),
        pltpu.VMEM((L, HID), _f32),
        pltpu.VMEM((L, HID), _f32),
        pltpu.VMEM((L,), _f32),
        pltpu.VMEM((L,), _f32),
        pltpu.VMEM((L, HID), _f32),
        pltpu.VMEM((L, HID), _f32),
    ])
def _sc_norm(outp_i, denp_i, bias_h, skip_i, x_o,
             b_v, o0_v, o1_v, d0_v, d1_v, sk_v, xb_v):
    wid = _wid()
    pltpu.sync_copy(bias_h, b_v)

    def row_chunk(k, _):
        rb = (wid * RPT + k) * L
        pltpu.sync_copy(outp_i.at[0, pl.ds(rb, L), :], o0_v)
        pltpu.sync_copy(outp_i.at[1, pl.ds(rb, L), :], o1_v)
        pltpu.sync_copy(denp_i.at[0, pl.ds(rb, L)], d0_v)
        pltpu.sync_copy(denp_i.at[1, pl.ds(rb, L)], d1_v)
        pltpu.sync_copy(skip_i.at[pl.ds(rb, L), :], sk_v)
        d0c = d0_v[...]
        d1c = d1_v[...]
        for r in range(L):
            d = d0c[r] + d1c[r] + 1e-16
            for c8 in range(HID // L):
                cs = pl.ds(c8 * L, L)
                v = (o0_v[r, cs] + o1_v[r, cs]) / d + b_v[cs] + sk_v[r, cs]
                xb_v[r, cs] = jnp.where(v > 0.0, v, jnp.exp(v) - 1.0)
        pltpu.sync_copy(xb_v, x_o.at[pl.ds(rb, L), :])
        return 0

    lax.fori_loop(0, RPT, row_chunk, 0)


def _mm_body(nw):
    def body(x_ref, p_ref, wx_ref, wp_ref, b_ref, *out_refs):
        x = x_ref[...]
        p = p_ref[...]
        for i in range(nw):
            acc = jnp.dot(x, wx_ref[i], preferred_element_type=_f32)
            acc = acc + jnp.dot(p, wp_ref[i], preferred_element_type=_f32)
            out_refs[i][...] = acc + b_ref[i, 0:1, :]
    return body


def _lin(x, posp, wxs, wps, bs, nw):
    bn = 2048
    return pl.pallas_call(
        _mm_body(nw),
        grid=(NP // bn,),
        in_specs=[
            pl.BlockSpec((bn, HID), lambda i: (i, 0)),
            pl.BlockSpec((bn, HID), lambda i: (i, 0)),
            pl.BlockSpec((nw, HID, HID), lambda i: (0, 0, 0)),
            pl.BlockSpec((nw, HID, HID), lambda i: (0, 0, 0)),
            pl.BlockSpec((nw, 8, HID), lambda i: (0, 0, 0)),
        ],
        out_specs=[pl.BlockSpec((bn, HID), lambda i: (i, 0))] * nw,
        out_shape=[jax.ShapeDtypeStruct((NP, HID), _f32)] * nw,
    )(x, posp, wxs, wps, bs)


def _pad_posw(w):
    return jnp.zeros((HID, HID), _f32).at[:3, :].set(w[:, HID:].T)


def _pad_bias(b):
    return jnp.zeros((8, HID), _f32).at[0, :b.shape[0]].set(b)


def kernel(latent, pos, edge_attr, params, edge_index):
    del edge_attr
    prm = params
    src_p = jnp.pad(edge_index[0], (0, EP - E))
    dst_p = jnp.pad(edge_index[1], (0, EP - E))
    lat_p = jnp.pad(latent.astype(_f32), ((0, NP - N), (0, 0)))
    pos_p = jnp.pad(pos.astype(_f32), ((0, NP - N), (0, HID - 3)))
    px = jnp.pad(pos[:, 0], (0, NP - N))
    py = jnp.pad(pos[:, 1], (0, NP - N))
    pz = jnp.pad(pos[:, 2], (0, NP - N))

    zero_w = jnp.zeros((1, HID, HID), _f32)
    x = _lin(lat_p, pos_p, prm['W0'].T[None], zero_w,
             _pad_bias(prm['b0'])[None], 1)[0]

    layers = [(prm['gat0'], prm['W1'], prm['b1']),
              (prm['gat1'], prm['W2'], prm['b2']),
              (prm['gat2'], prm['W3'], prm['b3'])]
    for gat, wk, bk in layers:
        wxs = jnp.stack([gat['Wl'][:, :HID].T, gat['Wr'][:, :HID].T,
                         wk[:, :HID].T])
        wps = jnp.stack([_pad_posw(gat['Wl']), _pad_posw(gat['Wr']),
                         _pad_posw(wk)])
        bss = jnp.stack([_pad_bias(gat['bl']), _pad_bias(gat['br']),
                         _pad_bias(bk)])
        xl, xr, skip = _lin(x, pos_p, wxs, wps, bss, 3)
        alpha, pmax = _sc_attn(xl, xr, px, py, pz, src_p, dst_p,
                               gat['att'], gat['We'][:, 0],
                               gat['We'][:, 1], gat['We'][:, 2])
        outp, denp = _sc_aggr(xl, src_p, dst_p, alpha, pmax)
        x = _sc_norm(outp, denp, gat['bias'], skip)

    w4x = jnp.zeros((HID, HID), _f32).at[:, :OUT].set(prm['W4'][:, :HID].T)
    w4p = jnp.zeros((HID, HID), _f32).at[:3, :OUT].set(prm['W4'][:, HID:].T)
    out = _lin(x, pos_p, w4x[None], w4p[None], _pad_bias(prm['b4'])[None],
               1)[0]
    return out[:N, :OUT]


# speedup vs baseline: 9.6129x; 4.1831x over previous
import functools

import jax
import jax.numpy as jnp
from jax import lax
from jax.experimental import pallas as pl
from jax.experimental.pallas import tpu as pltpu
from jax.experimental.pallas import tpu_sc as plsc

N = 10000
NP = 10240
E = 320000
EP = E + 128
HID = 128
HD2 = 144
OUT = 3
NC = 2
NS = 16
NW = NC * NS
L = 16
EW = E // NW
C = 80
NCH = EW // C
G = C // L
RPT = NP // NW // L
ZR = 32

_mesh = plsc.VectorSubcoreMesh(
    core_axis_name="c", subcore_axis_name="s", num_cores=NC, num_subcores=NS)
_params = pltpu.CompilerParams(needs_layout_passes=False)

_f32 = jnp.float32


def _wid():
    return lax.axis_index("c") * NS + lax.axis_index("s")


@functools.partial(
    pl.kernel,
    out_type=(jax.ShapeDtypeStruct((EP,), _f32),
              jax.ShapeDtypeStruct((NW * L,), _f32)),
    mesh=_mesh,
    compiler_params=_params,
    scratch_types=[
        pltpu.VMEM((NP,), _f32),
        pltpu.VMEM((NP,), _f32),
        pltpu.VMEM((NP,), _f32),
        pltpu.VMEM((HID,), _f32),
        pltpu.VMEM((HID,), _f32),
        pltpu.VMEM((HID,), _f32),
        pltpu.VMEM((HID,), _f32),
        pltpu.VMEM((2, C), jnp.int32),
        pltpu.VMEM((2, C), jnp.int32),
        pltpu.VMEM((2, C, HID), _f32),
        pltpu.VMEM((2, C, HID), _f32),
        pltpu.VMEM((2, C), _f32),
        pltpu.VMEM((L,), _f32),
        pltpu.SemaphoreType.DMA,
        pltpu.SemaphoreType.DMA,
        pltpu.SemaphoreType.DMA,
        pltpu.SemaphoreType.DMA,
        pltpu.SemaphoreType.DMA,
        pltpu.SemaphoreType.DMA,
    ])
def _sc_attn(xl, xr, px, py, pz, srcp, dstp, att_h, wex_h, wey_h, wez_h,
             alpha_o, pmax_o,
             px_v, py_v, pz_v, att_v, wex_v, wey_v, wez_v,
             se_v, de_v, xj_v, xi_v, al_v, mx_v,
             s_ei0, s_ei1, s_gat0, s_gat1, s_al0, s_al1):
    wid = _wid()
    base_e = wid * EW
    s_ei = (s_ei0, s_ei1)
    s_gat = (s_gat0, s_gat1)
    s_al = (s_al0, s_al1)
    pltpu.sync_copy(px, px_v)
    pltpu.sync_copy(py, py_v)
    pltpu.sync_copy(pz, pz_v)
    pltpu.sync_copy(att_h, att_v)
    pltpu.sync_copy(wex_h, wex_v)
    pltpu.sync_copy(wey_h, wey_v)
    pltpu.sync_copy(wez_h, wez_v)
    iot = lax.iota(jnp.int32, L)
    colv = [c * L + iot for c in range(HID // L)]
    lmask = [iot == lane for lane in range(L)]

    def start_ei(j, b):
        pltpu.async_copy(srcp.at[pl.ds(base_e + j * C, C)], se_v.at[b],
                         s_ei[b])
        pltpu.async_copy(dstp.at[pl.ds(base_e + j * C, C)], de_v.at[b],
                         s_ei[b])

    def wait_ei(b):
        pltpu.make_async_copy(srcp.at[pl.ds(base_e, C)], se_v.at[b],
                              s_ei[b]).wait()
        pltpu.make_async_copy(dstp.at[pl.ds(base_e, C)], de_v.at[b],
                              s_ei[b]).wait()

    def start_gat(j, b):
        del j
        pltpu.async_copy(xl.at[se_v.at[b]], xj_v.at[b], s_gat[b])
        pltpu.async_copy(xr.at[de_v.at[b]], xi_v.at[b], s_gat[b])

    def wait_gat(b):
        pltpu.make_async_copy(xl.at[se_v.at[b]], xj_v.at[b],
                              s_gat[b]).wait()
        pltpu.make_async_copy(xr.at[de_v.at[b]], xi_v.at[b],
                              s_gat[b]).wait()

    def wait_al(b):
        pltpu.make_async_copy(al_v.at[b], alpha_o.at[pl.ds(base_e, C)],
                              s_al[b]).wait()

    def process(j, b, maxv, last):
        b1 = 1 - b

        @pl.when(j >= 2)
        def _():
            wait_al(b)

        wait_gat(b)
        if not last:
            wait_ei(b1)
            start_gat(j + 1, b1)
        if not last:
            start_ei(j + 2, b)

        wxc = [wex_v[pl.ds(c * L, L)] for c in range(HID // L)]
        wyc = [wey_v[pl.ds(c * L, L)] for c in range(HID // L)]
        wzc = [wez_v[pl.ds(c * L, L)] for c in range(HID // L)]
        afc = [att_v[pl.ds(c * L, L)] for c in range(HID // L)]

        def g_step(g, maxv):
            gs = pl.ds(g * L, L)
            sg = se_v[b, gs]
            dg = de_v[b, gs]
            exv = plsc.load_gather(px_v, [dg]) - plsc.load_gather(px_v, [sg])
            eyv = plsc.load_gather(py_v, [dg]) - plsc.load_gather(py_v, [sg])
            ezv = plsc.load_gather(pz_v, [dg]) - plsc.load_gather(pz_v, [sg])
            alpha_g = jnp.zeros((L,), _f32)
            for lane in range(L):
                rowv = jnp.full((L,), 0, jnp.int32) + (g * L + lane)
                ex_s = exv[lane]
                ey_s = eyv[lane]
                ez_s = ezv[lane]
                acc = jnp.zeros((L,), _f32)
                for c in range(HID // L):
                    xj = plsc.load_gather(xj_v.at[b], [rowv, colv[c]])
                    xi = plsc.load_gather(xi_v.at[b], [rowv, colv[c]])
                    v = (xj + xi + wxc[c] * ex_s + wyc[c] * ey_s
                         + wzc[c] * ez_s)
                    acc = acc + afc[c] * jnp.maximum(v, 0.2 * v)
                alpha_g = jnp.where(lmask[lane], jnp.sum(acc), alpha_g)
            al_v[b, gs] = alpha_g
            return jnp.maximum(maxv, alpha_g)

        maxv = lax.fori_loop(0, G, g_step, maxv)
        pltpu.async_copy(al_v.at[b], alpha_o.at[pl.ds(base_e + j * C, C)],
                         s_al[b])
        return maxv

    start_ei(0, 0)
    wait_ei(0)
    start_gat(0, 0)
    start_ei(1, 1)

    def pair(i, maxv):
        j0 = i * 2
        maxv = process(j0, 0, maxv, False)
        maxv = process(j0 + 1, 1, maxv, False)
        return maxv

    maxv = lax.fori_loop(0, NCH // 2, pair,
                         jnp.full((L,), -3.4e38, _f32))
    maxv = process(NCH - 1, 0, maxv, True)
    wait_al(1)
    wait_al(0)
    wait_ei(1)
    mx_v[...] = maxv
    pltpu.sync_copy(mx_v, pmax_o.at[pl.ds(wid * L, L)])


@functools.partial(
    pl.kernel,
    out_type=(jax.ShapeDtypeStruct((NC, NP, HID), _f32),
              jax.ShapeDtypeStruct((NC, NP), _f32)),
    mesh=_mesh,
    compiler_params=_params,
    scratch_types=[
        pltpu.VMEM_SHARED((NP, HID), _f32),
        pltpu.VMEM_SHARED((NP,), _f32),
        pltpu.VMEM((NW * L,), _f32),
        pltpu.VMEM((2, C), jnp.int32),
        pltpu.VMEM((2, C), jnp.int32),
        pltpu.VMEM((2, C), jnp.int32),
        pltpu.VMEM((2, C, HID), _f32),
        pltpu.VMEM((2, C, HID), _f32),
        pltpu.VMEM((2, C), _f32),
        pltpu.VMEM((2, C), _f32),
        pltpu.VMEM((ZR, HID), _f32),
        pltpu.VMEM((NP // NS,), _f32),
        pltpu.SemaphoreType.DMA,
        pltpu.SemaphoreType.DMA,
        pltpu.SemaphoreType.DMA,
        pltpu.SemaphoreType.DMA,
        pltpu.SemaphoreType.DMA,
        pltpu.SemaphoreType.DMA,
        pltpu.SemaphoreType.DMA,
        pltpu.SemaphoreType.DMA,
        pltpu.SemaphoreType.DMA,
        pltpu.SemaphoreType.DMA,
    ])
def _sc_aggr(xl, srcp, dstp, alpha_i, pmax_i,
             outp_o, denp_o,
             out_sh, den_sh, pm_v, se_v, de_v, dsti_v, xjg_v, sc_v, ali_v,
             ex_v, z_v, dz_v,
             s_ei0, s_ei1, s_ali0, s_ali1, s_gat0, s_gat1, s_sc0, s_sc1,
             s_scx0, s_scx1):
    cid = lax.axis_index("c")
    sid = lax.axis_index("s")
    wid = cid * NS + sid
    base_e = wid * EW
    s_ei = (s_ei0, s_ei1)
    s_ali = (s_ali0, s_ali1)
    s_gat = (s_gat0, s_gat1)
    s_sc = (s_sc0, s_sc1)
    s_scx = (s_scx0, s_scx1)
    iot = lax.iota(jnp.int32, L)
    colv = [c * L + iot for c in range(HID // L)]
    zv = jnp.zeros((L,), _f32)
    for r in range(ZR):
        for c8 in range(HID // L):
            z_v[r, pl.ds(c8 * L, L)] = zv
    nds = NP // NS

    def dz_fill(i, _):
        dz_v[pl.ds(i * L, L)] = zv
        return 0

    lax.fori_loop(0, nds // L, dz_fill, 0)
    rb = sid * nds
    for t in range(nds // ZR):
        pltpu.sync_copy(z_v, out_sh.at[pl.ds(rb + t * ZR, ZR), :])
    pltpu.sync_copy(dz_v, den_sh.at[pl.ds(rb, nds)])
    plsc.subcore_barrier()

    pltpu.sync_copy(pmax_i, pm_v)
    mv = pm_v[pl.ds(0, L)]
    for i in range(1, NW):
        mv = jnp.maximum(mv, pm_v[pl.ds(i * L, L)])
    m_glob = jnp.max(mv)

    def start_ei(j, b):
        pltpu.async_copy(srcp.at[pl.ds(base_e + j * C, C)], se_v.at[b],
                         s_ei[b])
        pltpu.async_copy(dstp.at[pl.ds(base_e + j * C, C)], de_v.at[b],
                         s_ei[b])
        pltpu.async_copy(alpha_i.at[pl.ds(base_e + j * C, C)], ali_v.at[b],
                         s_ali[b])

    def wait_ei(b):
        pltpu.make_async_copy(srcp.at[pl.ds(base_e, C)], se_v.at[b],
                              s_ei[b]).wait()
        pltpu.make_async_copy(dstp.at[pl.ds(base_e, C)], de_v.at[b],
                              s_ei[b]).wait()
        pltpu.make_async_copy(alpha_i.at[pl.ds(base_e, C)], ali_v.at[b],
                              s_ali[b]).wait()

    def start_gat(j, b):
        del j
        pltpu.async_copy(xl.at[se_v.at[b]], xjg_v.at[b], s_gat[b])

    def wait_gat(b):
        pltpu.make_async_copy(xl.at[se_v.at[b]], xjg_v.at[b],
                              s_gat[b]).wait()

    def wait_sc(b):
        pltpu.make_async_copy(sc_v.at[b], out_sh.at[dsti_v.at[b]],
                              s_sc[b]).wait()
        pltpu.make_async_copy(ex_v.at[b], den_sh.at[dsti_v.at[b]],
                              s_scx[b]).wait()

    def process(j, b, last):
        b1 = 1 - b

        @pl.when(j >= 2)
        def _():
            wait_sc(b)

        wait_gat(b)
        if not last:
            wait_ei(b1)
            start_gat(j + 1, b1)
        exgs = []
        for g in range(G):
            gs = pl.ds(g * L, L)
            dsti_v[b, gs] = de_v[b, gs]
            eg = jnp.exp(ali_v[b, gs] - m_glob)
            exgs.append(eg)
            ex_v[b, gs] = eg
        if not last:
            start_ei(j + 2, b)
        pltpu.async_copy(ex_v.at[b], den_sh.at[dsti_v.at[b]], s_scx[b],
                         add=True)

        def g_step(g, carry):
            exv = ex_v[b, pl.ds(g * L, L)]
            for lane in range(L):
                rowv = jnp.full((L,), 0, jnp.int32) + (g * L + lane)
                ex_s = exv[lane]
                for c in range(HID // L):
                    xv = plsc.load_gather(xjg_v.at[b], [rowv, colv[c]])
                    plsc.store_scatter(sc_v.at[b], [rowv, colv[c]],
                                       xv * ex_s)
            return carry

        lax.fori_loop(0, G, g_step, 0)
        pltpu.async_copy(sc_v.at[b], out_sh.at[dsti_v.at[b]], s_sc[b],
                         add=True)

    start_ei(0, 0)
    wait_ei(0)
    start_gat(0, 0)
    start_ei(1, 1)

    def pair(i, _):
        j0 = i * 2
        process(j0, 0, False)
        process(j0 + 1, 1, False)
        return 0

    lax.fori_loop(0, NCH // 2, pair, 0)
    process(NCH - 1, 0, True)
    wait_sc(1)
    wait_sc(0)
    wait_ei(1)
    plsc.subcore_barrier()
    for t in range(nds // ZR):
        pltpu.sync_copy(out_sh.at[pl.ds(rb + t * ZR, ZR), :],
                        outp_o.at[cid, pl.ds(rb + t * ZR, ZR), :])
    pltpu.sync_copy(den_sh.at[pl.ds(rb, nds)], denp_o.at[cid, pl.ds(rb, nds)])


@functools.partial(
    pl.kernel,
    out_type=jax.ShapeDtypeStruct((NP, HID), _f32),
    mesh=_mesh,
    compiler_params=_params,
    scratch_types=[
        pltpu.VMEM((HID,), _f32),
        pltpu.VMEM((L, HID), _f32),
        pltpu.VMEM((L, HID), _f32),
        pltpu.VMEM((L,), _f32),
        pltpu.VMEM((L,), _f32),
        pltpu.VMEM((L, HID), _f32),
        pltpu.VMEM((L, HID), _f32),
    ])
def _sc_norm(outp_i, denp_i, bias_h, skip_i, x_o,
             b_v, o0_v, o1_v, d0_v, d1_v, sk_v, xb_v):
    wid = _wid()
    pltpu.sync_copy(bias_h, b_v)

    def row_chunk(k, _):
        rb = (wid * RPT + k) * L
        pltpu.sync_copy(outp_i.at[0, pl.ds(rb, L), :], o0_v)
        pltpu.sync_copy(outp_i.at[1, pl.ds(rb, L), :], o1_v)
        pltpu.sync_copy(denp_i.at[0, pl.ds(rb, L)], d0_v)
        pltpu.sync_copy(denp_i.at[1, pl.ds(rb, L)], d1_v)
        pltpu.sync_copy(skip_i.at[pl.ds(rb, L), :], sk_v)
        d0c = d0_v[...]
        d1c = d1_v[...]
        for r in range(L):
            d = d0c[r] + d1c[r] + 1e-16
            for c8 in range(HID // L):
                cs = pl.ds(c8 * L, L)
                v = (o0_v[r, cs] + o1_v[r, cs]) / d + b_v[cs] + sk_v[r, cs]
                xb_v[r, cs] = jnp.where(v > 0.0, v, jnp.exp(v) - 1.0)
        pltpu.sync_copy(xb_v, x_o.at[pl.ds(rb, L), :])
        return 0

    lax.fori_loop(0, RPT, row_chunk, 0)


def _mm_body(nw):
    def body(x_ref, p_ref, wx_ref, wp_ref, b_ref, *out_refs):
        x = x_ref[...]
        p = p_ref[...]
        for i in range(nw):
            acc = jnp.dot(x, wx_ref[i], preferred_element_type=_f32)
            acc = acc + jnp.dot(p, wp_ref[i], preferred_element_type=_f32)
            out_refs[i][...] = acc + b_ref[i, 0:1, :]
    return body


def _lin(x, posp, wxs, wps, bs, nw):
    bn = 2048
    return pl.pallas_call(
        _mm_body(nw),
        grid=(NP // bn,),
        in_specs=[
            pl.BlockSpec((bn, HID), lambda i: (i, 0)),
            pl.BlockSpec((bn, HID), lambda i: (i, 0)),
            pl.BlockSpec((nw, HID, HID), lambda i: (0, 0, 0)),
            pl.BlockSpec((nw, HID, HID), lambda i: (0, 0, 0)),
            pl.BlockSpec((nw, 8, HID), lambda i: (0, 0, 0)),
        ],
        out_specs=[pl.BlockSpec((bn, HID), lambda i: (i, 0))] * nw,
        out_shape=[jax.ShapeDtypeStruct((NP, HID), _f32)] * nw,
    )(x, posp, wxs, wps, bs)


def _pad_posw(w):
    return jnp.zeros((HID, HID), _f32).at[:3, :].set(w[:, HID:].T)


def _pad_bias(b):
    return jnp.zeros((8, HID), _f32).at[0, :b.shape[0]].set(b)


def kernel(latent, pos, edge_attr, params, edge_index):
    del edge_attr
    prm = params
    src_p = jnp.pad(edge_index[0], (0, EP - E))
    dst_p = jnp.pad(edge_index[1], (0, EP - E))
    lat_p = jnp.pad(latent.astype(_f32), ((0, NP - N), (0, 0)))
    pos_p = jnp.pad(pos.astype(_f32), ((0, NP - N), (0, HID - 3)))
    px = jnp.pad(pos[:, 0], (0, NP - N))
    py = jnp.pad(pos[:, 1], (0, NP - N))
    pz = jnp.pad(pos[:, 2], (0, NP - N))

    zero_w = jnp.zeros((1, HID, HID), _f32)
    x = _lin(lat_p, pos_p, prm['W0'].T[None], zero_w,
             _pad_bias(prm['b0'])[None], 1)[0]

    layers = [(prm['gat0'], prm['W1'], prm['b1']),
              (prm['gat1'], prm['W2'], prm['b2']),
              (prm['gat2'], prm['W3'], prm['b3'])]
    for gat, wk, bk in layers:
        wxs = jnp.stack([gat['Wl'][:, :HID].T, gat['Wr'][:, :HID].T,
                         wk[:, :HID].T])
        wps = jnp.stack([_pad_posw(gat['Wl']), _pad_posw(gat['Wr']),
                         _pad_posw(wk)])
        bss = jnp.stack([_pad_bias(gat['bl']), _pad_bias(gat['br']),
                         _pad_bias(bk)])
        xl, xr, skip = _lin(x, pos_p, wxs, wps, bss, 3)
        alpha, pmax = _sc_attn(xl, xr, px, py, pz, src_p, dst_p,
                               gat['att'], gat['We'][:, 0],
                               gat['We'][:, 1], gat['We'][:, 2])
        outp, denp = _sc_aggr(xl, src_p, dst_p, alpha, pmax)
        x = _sc_norm(outp, denp, gat['bias'], skip)

    w4x = jnp.zeros((HID, HID), _f32).at[:, :OUT].set(prm['W4'][:, :HID].T)
    w4p = jnp.zeros((HID, HID), _f32).at[:3, :OUT].set(prm['W4'][:, HID:].T)
    out = _lin(x, pos_p, w4x[None], w4p[None], _pad_bias(prm['b4'])[None],
               1)[0]
    return out[:N, :OUT]
